# baseline (speedup 1.0000x reference)
import functools

import jax
import jax.numpy as jnp
from jax.experimental import pallas as pl
from jax.experimental.pallas import tpu as pltpu


def _gcn_kernel(x_ref, w1_ref, w2_ref, a0_ref, a1_ref, o_ref,
                z_ref, h_ref, r_ref, zb_ref, *, bm):
    p = pl.program_id(0)
    i = pl.program_id(1)
    hb = bm // 2

    @pl.when((p == 0) & (i == 0))
    def _():
        z_ref[...] = jnp.dot(x_ref[...], w1_ref[...],
                             preferred_element_type=jnp.float32)

    @pl.when((p == 1) & (i == 0))
    def _():
        z2 = jnp.dot(h_ref[...], w2_ref[...],
                     preferred_element_type=jnp.float32)
        z_ref[...] = z2
        zb_ref[...] = z2.astype(jnp.bfloat16)

    @pl.when(p == 0)
    def _():
        acc0 = jnp.dot(a0_ref[...], z_ref[...],
                       preferred_element_type=jnp.float32)
        acc1 = jnp.dot(a1_ref[...], z_ref[...],
                       preferred_element_type=jnp.float32)
        nblk = pl.num_programs(1)
        j = nblk - 1 - i
        h_ref[pl.ds(j * bm, hb), :] = acc0
        h_ref[pl.ds(j * bm + hb, hb), :] = acc1

        @pl.when(j == 1)
        def _():
            r_ref[pl.ds(0, hb), :] = a0_ref[...].astype(jnp.bfloat16)
            r_ref[pl.ds(hb, hb), :] = a1_ref[...].astype(jnp.bfloat16)

    @pl.when((p == 1) & (i != 1))
    def _():
        o_ref[pl.ds(0, hb), :] = jnp.dot(
            a0_ref[...], z_ref[...], preferred_element_type=jnp.float32)
        o_ref[pl.ds(hb, hb), :] = jnp.dot(
            a1_ref[...], z_ref[...], preferred_element_type=jnp.float32)

    @pl.when((p == 1) & (i == 1))
    def _():
        o_ref[...] = jnp.dot(r_ref[...], zb_ref[...],
                             preferred_element_type=jnp.float32)


def kernel(X, A_hat, W1, W2):
    n = A_hat.shape[0]
    d = W1.shape[1]
    bm = 400 if n % 400 == 0 else n
    nblk = n // bm

    def blk(p, i):
        fwd = (i > 1).astype(jnp.int32) * i
        return (1 - p) * (nblk - 1 - i) + p * fwd

    return pl.pallas_call(
        functools.partial(_gcn_kernel, bm=bm),
        grid=(2, nblk),
        in_specs=[
            pl.BlockSpec((n, d), lambda p, i: (0, 0)),
            pl.BlockSpec((d, d), lambda p, i: (0, 0)),
            pl.BlockSpec((d, d), lambda p, i: (0, 0)),
            pl.BlockSpec((bm // 2, n), lambda p, i: (2 * blk(p, i), 0)),
            pl.BlockSpec((bm // 2, n), lambda p, i: (2 * blk(p, i) + 1, 0)),
        ],
        out_specs=pl.BlockSpec((bm, d), lambda p, i: (p * i, 0)),
        out_shape=jax.ShapeDtypeStruct((n, d), jnp.float32),
        scratch_shapes=[
            pltpu.VMEM((n, d), jnp.float32),
            pltpu.VMEM((n, d), jnp.float32),
            pltpu.VMEM((bm, n), jnp.bfloat16),
            pltpu.VMEM((n, d), jnp.bfloat16),
        ],
    )(X, W1, W2, A_hat, A_hat)


# speedup vs baseline: 1.0111x; 1.0111x over previous
import functools

import jax
import jax.numpy as jnp
from jax.experimental import pallas as pl
from jax.experimental.pallas import tpu as pltpu


def _gcn_kernel(x_ref, w1_ref, w2_ref, a_ref, o_ref, z_ref, h_ref, r_ref,
                zb_ref, *, bm):
    p = pl.program_id(0)
    i = pl.program_id(1)
    nblk = pl.num_programs(1)

    @pl.when((p == 0) & (i == 0))
    def _():
        z_ref[...] = jnp.dot(x_ref[...], w1_ref[...],
                             preferred_element_type=jnp.float32)

    @pl.when((p == 1) & (i == 0))
    def _():
        z2 = jnp.dot(h_ref[...], w2_ref[...],
                     preferred_element_type=jnp.float32)
        z_ref[...] = z2
        zb_ref[...] = z2.astype(jnp.bfloat16)

    @pl.when(p == 0)
    def _():
        acc = jnp.dot(a_ref[...], z_ref[...],
                      preferred_element_type=jnp.float32)
        j = nblk - 1 - i
        h_ref[pl.ds(j * bm, bm), :] = acc

        @pl.when(j == 1)
        def _():
            r_ref[...] = a_ref[...].astype(jnp.bfloat16)

    @pl.when((p == 1) & (i != nblk - 1))
    def _():
        o_ref[...] = jnp.dot(a_ref[...], z_ref[...],
                             preferred_element_type=jnp.float32)

    @pl.when((p == 1) & (i == nblk - 1))
    def _():
        o_ref[...] = jnp.dot(r_ref[...], zb_ref[...],
                             preferred_element_type=jnp.float32)


def kernel(X, A_hat, W1, W2):
    n = A_hat.shape[0]
    d = W1.shape[1]
    bm = 400 if n % 400 == 0 else n
    nblk = n // bm

    def p1_out(i):
        return ((i > 0).astype(jnp.int32) * (i + 1)
                - (i == nblk - 1).astype(jnp.int32) * (nblk - 1))

    def a_map(p, i):
        fwd = ((i > 0).astype(jnp.int32) * (i + 1)
               - (i == nblk - 1).astype(jnp.int32))
        return ((1 - p) * (nblk - 1 - i) + p * fwd, 0)

    return pl.pallas_call(
        functools.partial(_gcn_kernel, bm=bm),
        grid=(2, nblk),
        in_specs=[
            pl.BlockSpec((n, d), lambda p, i: (0, 0)),
            pl.BlockSpec((d, d), lambda p, i: (0, 0)),
            pl.BlockSpec((d, d), lambda p, i: (0, 0)),
            pl.BlockSpec((bm, n), a_map),
        ],
        out_specs=pl.BlockSpec((bm, d), lambda p, i: (p * p1_out(i), 0)),
        out_shape=jax.ShapeDtypeStruct((n, d), jnp.float32),
        scratch_shapes=[
            pltpu.VMEM((n, d), jnp.float32),
            pltpu.VMEM((n, d), jnp.float32),
            pltpu.VMEM((bm, n), jnp.bfloat16),
            pltpu.VMEM((n, d), jnp.bfloat16),
        ],
    )(X, W1, W2, A_hat)
